# Initial kernel scaffold; baseline (speedup 1.0000x reference)
import jax
import jax.numpy as jnp
from jax.experimental import pallas as pl


def kernel(hour, weekday, day, month, E_hour, E_weekday, E_day, E_month):
    raise NotImplementedError("write your pallas kernel here")


# speedup vs baseline: 8.8169x; 8.8169x over previous
import jax
import jax.numpy as jnp
from jax import lax
from jax.experimental import pallas as pl
from jax.experimental.pallas import tpu as pltpu
from jax.experimental.pallas import tpu_sc as plsc

B, T, D = 4096, 200, 64
N = B * T
NC, NS = 2, 16
NW = NC * NS
ROWS_PER_W = N // NW
CHUNK = 512
NCHUNKS = ROWS_PER_W // CHUNK

N1 = 24 * 7
N2 = 32 * 13


def _sc_body(h_hbm, w_hbm, d_hbm, m_hbm,
             eh_hbm, ew_hbm, ed_hbm, em_hbm,
             out_hbm,
             eh_v, ew_v, ed_v, em_v,
             t1_v, t2_v,
             h_v, w_v, d_v, m_v, i1_v, i2_v,
             out_v, sem):
    wid = lax.axis_index("s") * NC + lax.axis_index("c")
    base = wid * ROWS_PER_W

    pltpu.sync_copy(eh_hbm, eh_v)
    pltpu.sync_copy(ew_hbm, ew_v)
    pltpu.sync_copy(ed_hbm, ed_v)
    pltpu.sync_copy(em_hbm, em_v)

    def build1(k, _):
        h = k // 7
        w = k - h * 7
        for j in range(D // 16):
            t1_v[pl.ds(k * D + 16 * j, 16)] = (
                eh_v[pl.ds(h * D + 16 * j, 16)] + ew_v[pl.ds(w * D + 16 * j, 16)])
        return _

    lax.fori_loop(0, N1, build1, None)

    def build2(k, _):
        d = k // 13
        m = k - d * 13
        for j in range(D // 16):
            t2_v[pl.ds(k * D + 16 * j, 16)] = (
                ed_v[pl.ds(d * D + 16 * j, 16)] + em_v[pl.ds(m * D + 16 * j, 16)])
        return _

    lax.fori_loop(0, N2, build2, None)

    def chunk_body(g, _):
        start = base + g * CHUNK
        pltpu.sync_copy(h_hbm.at[pl.ds(start, CHUNK)], h_v)
        pltpu.sync_copy(w_hbm.at[pl.ds(start, CHUNK)], w_v)
        pltpu.sync_copy(d_hbm.at[pl.ds(start, CHUNK)], d_v)
        pltpu.sync_copy(m_hbm.at[pl.ds(start, CHUNK)], m_v)

        def fuse(v, _):
            s = pl.ds(16 * v, 16)
            i1_v[s] = h_v[s] * 7 + w_v[s]
            i2_v[s] = d_v[s] * 13 + m_v[s]
            return _

        lax.fori_loop(0, CHUNK // 16, fuse, None)

        def row16(q, _):
            v1 = i1_v[pl.ds(q * 16, 16)] * D
            v2 = i2_v[pl.ds(q * 16, 16)] * D
            for l in range(16):
                a = v1[l]
                b = v2[l]
                o = (q * 16 + l) * D
                for j in range(D // 16):
                    out_v[pl.ds(o + 16 * j, 16)] = (
                        t1_v[pl.ds(a + 16 * j, 16)] + t2_v[pl.ds(b + 16 * j, 16)])
            return _

        lax.fori_loop(0, CHUNK // 16, row16, None)

        pltpu.sync_copy(out_v, out_hbm.at[pl.ds(start * D, CHUNK * D)])
        return _

    lax.fori_loop(0, NCHUNKS, chunk_body, None)


@jax.jit
def kernel(hour, weekday, day, month, E_hour, E_weekday, E_day, E_month):
    mesh = plsc.VectorSubcoreMesh(core_axis_name="c", subcore_axis_name="s")
    run = pl.kernel(
        _sc_body,
        out_type=jax.ShapeDtypeStruct((N * D,), jnp.float32),
        mesh=mesh,
        scratch_types=[
            pltpu.VMEM((24 * D,), jnp.float32),
            pltpu.VMEM((7 * D,), jnp.float32),
            pltpu.VMEM((32 * D,), jnp.float32),
            pltpu.VMEM((13 * D,), jnp.float32),
            pltpu.VMEM((N1 * D,), jnp.float32),
            pltpu.VMEM((N2 * D,), jnp.float32),
            pltpu.VMEM((CHUNK,), jnp.int32),
            pltpu.VMEM((CHUNK,), jnp.int32),
            pltpu.VMEM((CHUNK,), jnp.int32),
            pltpu.VMEM((CHUNK,), jnp.int32),
            pltpu.VMEM((CHUNK,), jnp.int32),
            pltpu.VMEM((CHUNK,), jnp.int32),
            pltpu.VMEM((CHUNK * D,), jnp.float32),
            pltpu.SemaphoreType.DMA,
        ],
    )
    out = run(hour.reshape(N), weekday.reshape(N), day.reshape(N),
              month.reshape(N),
              E_hour.reshape(24 * D), E_weekday.reshape(7 * D),
              E_day.reshape(32 * D), E_month.reshape(13 * D))
    return out.reshape(B, T, D)


# speedup vs baseline: 10.2028x; 1.1572x over previous
import jax
import jax.numpy as jnp
from jax import lax
from jax.experimental import pallas as pl
from jax.experimental.pallas import tpu as pltpu
from jax.experimental.pallas import tpu_sc as plsc

B, T, D = 4096, 200, 64
N = B * T
NC, NS = 2, 16
NW = NC * NS
ROWS_PER_W = N // NW
CHUNK = 256
NCHUNKS = ROWS_PER_W // CHUNK

N1 = 24 * 7
N2 = 32 * 13


def _sc_body(h_hbm, w_hbm, d_hbm, m_hbm,
             eh_hbm, ew_hbm, ed_hbm, em_hbm,
             out_hbm,
             eh_v, ew_v, ed_v, em_v,
             t1_v, t2_v,
             h0, w0, d0, m0, h1, w1, d1, m1,
             out0, out1,
             semi0, semi1, semo0, semo1, semt):
    wid = lax.axis_index("s") * NC + lax.axis_index("c")
    base = wid * ROWS_PER_W

    idx_srcs = (h_hbm, w_hbm, d_hbm, m_hbm)
    idx_bufs = ((h0, w0, d0, m0), (h1, w1, d1, m1))
    outs = (out0, out1)
    semis = (semi0, semi1)
    semos = (semo0, semo1)

    def start_idx(g, p):
        st = base + g * CHUNK
        for src, dst in zip(idx_srcs, idx_bufs[p]):
            pltpu.async_copy(src.at[pl.ds(st, CHUNK)], dst, semis[p])

    def wait_idx(p):
        for src, dst in zip(idx_srcs, idx_bufs[p]):
            pltpu.make_async_copy(src.at[pl.ds(0, CHUNK)], dst, semis[p]).wait()

    start_idx(0, 0)

    for src, dst in zip((eh_hbm, ew_hbm, ed_hbm, em_hbm),
                        (eh_v, ew_v, ed_v, em_v)):
        pltpu.async_copy(src, dst, semt)
    for src, dst in zip((eh_hbm, ew_hbm, ed_hbm, em_hbm),
                        (eh_v, ew_v, ed_v, em_v)):
        pltpu.make_async_copy(src, dst, semt).wait()

    def build1(k, _):
        h = k // 7
        w = k - h * 7
        for j in range(D // 16):
            t1_v[pl.ds(k * D + 16 * j, 16)] = (
                eh_v[pl.ds(h * D + 16 * j, 16)] + ew_v[pl.ds(w * D + 16 * j, 16)])
        return _

    lax.fori_loop(0, N1, build1, None)

    def build2(k, _):
        d = k // 13
        m = k - d * 13
        for j in range(D // 16):
            t2_v[pl.ds(k * D + 16 * j, 16)] = (
                ed_v[pl.ds(d * D + 16 * j, 16)] + em_v[pl.ds(m * D + 16 * j, 16)])
        return _

    lax.fori_loop(0, N2, build2, None)

    def outer(gg, _):
        for p in range(2):
            g = gg * 2 + p

            @pl.when(g + 1 < NCHUNKS)
            def _prefetch():
                start_idx(g + 1, 1 - p)

            wait_idx(p)

            @pl.when(g >= 2)
            def _reclaim():
                pltpu.make_async_copy(
                    outs[p], out_hbm.at[pl.ds(0, CHUNK * D)], semos[p]).wait()

            hv, wv, dv, mv = idx_bufs[p]
            ov = outs[p]

            def row16(q, c):
                s = pl.ds(q * 16, 16)
                v1 = (hv[s] * 7 + wv[s]) * D
                v2 = (dv[s] * 13 + mv[s]) * D
                for l in range(16):
                    a = v1[l]
                    b = v2[l]
                    o = (q * 16 + l) * D
                    for j in range(D // 16):
                        ov[pl.ds(o + 16 * j, 16)] = (
                            t1_v[pl.ds(a + 16 * j, 16)]
                            + t2_v[pl.ds(b + 16 * j, 16)])
                return c

            lax.fori_loop(0, CHUNK // 16, row16, None)

            st = base + g * CHUNK
            pltpu.async_copy(ov, out_hbm.at[pl.ds(st * D, CHUNK * D)], semos[p])
        return _

    lax.fori_loop(0, NCHUNKS // 2, outer, None)

    for p in range(2):
        pltpu.make_async_copy(
            outs[p], out_hbm.at[pl.ds(0, CHUNK * D)], semos[p]).wait()


@jax.jit
def kernel(hour, weekday, day, month, E_hour, E_weekday, E_day, E_month):
    mesh = plsc.VectorSubcoreMesh(core_axis_name="c", subcore_axis_name="s")
    run = pl.kernel(
        _sc_body,
        out_type=jax.ShapeDtypeStruct((N * D,), jnp.float32),
        mesh=mesh,
        scratch_types=[
            pltpu.VMEM((24 * D,), jnp.float32),
            pltpu.VMEM((7 * D,), jnp.float32),
            pltpu.VMEM((32 * D,), jnp.float32),
            pltpu.VMEM((13 * D,), jnp.float32),
            pltpu.VMEM((N1 * D,), jnp.float32),
            pltpu.VMEM((N2 * D,), jnp.float32),
            pltpu.VMEM((CHUNK,), jnp.int32),
            pltpu.VMEM((CHUNK,), jnp.int32),
            pltpu.VMEM((CHUNK,), jnp.int32),
            pltpu.VMEM((CHUNK,), jnp.int32),
            pltpu.VMEM((CHUNK,), jnp.int32),
            pltpu.VMEM((CHUNK,), jnp.int32),
            pltpu.VMEM((CHUNK,), jnp.int32),
            pltpu.VMEM((CHUNK,), jnp.int32),
            pltpu.VMEM((CHUNK * D,), jnp.float32),
            pltpu.VMEM((CHUNK * D,), jnp.float32),
            pltpu.SemaphoreType.DMA,
            pltpu.SemaphoreType.DMA,
            pltpu.SemaphoreType.DMA,
            pltpu.SemaphoreType.DMA,
            pltpu.SemaphoreType.DMA,
        ],
    )
    out = run(hour.reshape(N), weekday.reshape(N), day.reshape(N),
              month.reshape(N),
              E_hour.reshape(24 * D), E_weekday.reshape(7 * D),
              E_day.reshape(32 * D), E_month.reshape(13 * D))
    return out.reshape(B, T, D)
